# baseline (speedup 1.0000x reference)
import jax
import jax.numpy as jnp
from jax.experimental import pallas as pl
from jax.experimental.pallas import tpu as pltpu

_GRID = 2
_X_ROWS = 10000 // _GRID
_E_COLS = 320000 // _GRID


def _copy_body(x_ref, e_ref, ox_ref, oe_ref):
    ox_ref[...] = x_ref[...]
    oe_ref[...] = e_ref[...]


def kernel(x, edge_index, edge_attr):
    del edge_index
    e_t = edge_attr.T
    out_x, out_e_t = pl.pallas_call(
        _copy_body,
        grid=(_GRID,),
        in_specs=[
            pl.BlockSpec((_X_ROWS, 128), lambda i: (i, 0)),
            pl.BlockSpec((16, _E_COLS), lambda i: (0, i)),
        ],
        out_specs=[
            pl.BlockSpec((_X_ROWS, 128), lambda i: (i, 0)),
            pl.BlockSpec((16, _E_COLS), lambda i: (0, i)),
        ],
        out_shape=[
            jax.ShapeDtypeStruct(x.shape, x.dtype),
            jax.ShapeDtypeStruct(e_t.shape, e_t.dtype),
        ],
        compiler_params=pltpu.CompilerParams(
            dimension_semantics=("parallel",),
        ),
    )(x, e_t)
    return (out_x, out_e_t.T)
